# baseline (speedup 1.0000x reference)
import jax
import jax.numpy as jnp
from jax.experimental import pallas as pl
from jax.experimental.pallas import tpu as pltpu

_STATE = 4
_ACT = 6
_W = 21
_C0, _C1, _C2, _C3 = 64, 32, 64, 64
_M1 = 19 * _W + 20
_M2 = 8 * _W + 9
_S = 4
_G = 4


def _feat_kernel(x_ref, w1_ref, b1_ref, w2_ref, b2_ref, w3_ref, b3_ref,
                 o_ref, xp_ref, y1_ref, buf2_ref, y2_ref, buf3_ref):
    for g in range(_G):
        for s in range(_S):
            xp_ref[g, :, pl.ds(s * _C0, _C0)] = x_ref[g * _S + s]
    for g in range(_G):
        acc = None
        for p in range(2):
            for q in range(2):
                xs = xp_ref[g, pl.ds(p * _W + q, _M1), :]
                c = jnp.dot(xs, w1_ref[p * 2 + q],
                            preferred_element_type=jnp.float32)
                acc = c if acc is None else acc + c
        y1_ref[g] = jnp.maximum(acc + b1_ref[...], 0.0)

        for p in range(4):
            for q in range(4):
                t = p * 4 + q
                buf2_ref[g, :, pl.ds(t * _S * _C1, _S * _C1)] = (
                    y1_ref[g, pl.ds(p * _W + q, _M2, stride=2), :]
                    .astype(jnp.bfloat16))
        y2_ref[g] = jnp.maximum(
            jnp.dot(buf2_ref[g], w2_ref[...],
                    preferred_element_type=jnp.float32) + b2_ref[...], 0.0)

        for p in range(3):
            for q in range(3):
                t = p * 3 + q
                for oh in range(7):
                    buf3_ref[g, pl.ds(oh * 7, 7),
                             pl.ds(t * _S * _C2, _S * _C2)] = (
                        y2_ref[g, pl.ds((oh + p) * _W + q, 7), :]
                        .astype(jnp.bfloat16))
        z = jnp.maximum(
            jnp.dot(buf3_ref[g], w3_ref[...],
                    preferred_element_type=jnp.float32) + b3_ref[...], 0.0)
        for s in range(_S):
            o_ref[g * _S + s] = z[:, s * _C3:(s + 1) * _C3]


def _head_kernel(x_ref, wfc_ref, bfc_ref, wh_ref, bh_ref, logp_ref, val_ref):
    feat = jnp.dot(x_ref[...], wfc_ref[...],
                   preferred_element_type=jnp.float32)
    feat = jnp.maximum(feat + bfc_ref[...], 0.0)
    heads = jnp.dot(feat, wh_ref[...],
                    preferred_element_type=jnp.float32) + bh_ref[...]
    logits = heads[:, :_ACT]
    m = jnp.max(logits, axis=1, keepdims=True)
    z = logits - m
    lse = jnp.log(jnp.sum(jnp.exp(z), axis=1, keepdims=True))
    logp_ref[...] = z - lse
    val_ref[...] = heads[:, _ACT:]


def _block_diag(w):
    taps, cin, cout = w.shape
    eye = jnp.eye(_S, dtype=w.dtype)
    wbd = w[:, None, :, None, :] * eye[None, :, None, :, None]
    return wbd.reshape(taps * _S * cin, _S * cout)


def kernel(w1, b1, w2, b2, w3, b3, wfc, bfc, wh, bh, x):
    n = x.shape[0]
    xb = x.astype(jnp.bfloat16)
    x1 = xb.reshape(n, _STATE, _W, 4, _W, 4).transpose(0, 2, 4, 1, 3, 5)
    x1 = x1.reshape(n, _W * _W, _C0)

    w1bd = _block_diag(w1).reshape(4, _S * _C0, _S * _C1).astype(jnp.bfloat16)
    w2bd = _block_diag(w2).astype(jnp.bfloat16)
    w3bd = _block_diag(w3).astype(jnp.bfloat16)
    b1t = jnp.tile(b1, (1, _S))
    b2t = jnp.tile(b2, (1, _S))
    b3t = jnp.tile(b3, (1, _S))

    feat = pl.pallas_call(
        _feat_kernel,
        out_shape=jax.ShapeDtypeStruct((n, 49, _C3), jnp.float32),
        grid=(n // (_G * _S),),
        in_specs=[
            pl.BlockSpec((_G * _S, _W * _W, _C0), lambda i: (i, 0, 0)),
            pl.BlockSpec((4, _S * _C0, _S * _C1), lambda i: (0, 0, 0)),
            pl.BlockSpec((1, _S * _C1), lambda i: (0, 0)),
            pl.BlockSpec((16 * _S * _C1, _S * _C2), lambda i: (0, 0)),
            pl.BlockSpec((1, _S * _C2), lambda i: (0, 0)),
            pl.BlockSpec((9 * _S * _C2, _S * _C3), lambda i: (0, 0)),
            pl.BlockSpec((1, _S * _C3), lambda i: (0, 0)),
        ],
        out_specs=pl.BlockSpec((_G * _S, 49, _C3), lambda i: (i, 0, 0)),
        scratch_shapes=[
            pltpu.VMEM((_G, _W * _W, _S * _C0), jnp.bfloat16),
            pltpu.VMEM((_G, _M1, _S * _C1), jnp.float32),
            pltpu.VMEM((_G, _M2, 16 * _S * _C1), jnp.bfloat16),
            pltpu.VMEM((_G, _M2, _S * _C2), jnp.float32),
            pltpu.VMEM((_G, 49, 9 * _S * _C2), jnp.bfloat16),
        ],
        compiler_params=pltpu.CompilerParams(
            dimension_semantics=("parallel",),
            vmem_limit_bytes=48 * 1024 * 1024),
    )(x1, w1bd, b1t, w2bd, b2t, w3bd, b3t)

    flat = feat.reshape(n, 49 * _C3)

    h = wfc.shape[1]
    a1 = wh.shape[1]
    tm = 128
    logp, val = pl.pallas_call(
        _head_kernel,
        out_shape=(jax.ShapeDtypeStruct((n, a1 - 1), jnp.float32),
                   jax.ShapeDtypeStruct((n, 1), jnp.float32)),
        grid=(pl.cdiv(n, tm),),
        in_specs=[
            pl.BlockSpec((tm, 49 * _C3), lambda i: (i, 0)),
            pl.BlockSpec((49 * _C3, h), lambda i: (0, 0)),
            pl.BlockSpec((1, h), lambda i: (0, 0)),
            pl.BlockSpec((h, a1), lambda i: (0, 0)),
            pl.BlockSpec((1, a1), lambda i: (0, 0)),
        ],
        out_specs=(pl.BlockSpec((tm, a1 - 1), lambda i: (i, 0)),
                   pl.BlockSpec((tm, 1), lambda i: (i, 0))),
        compiler_params=pltpu.CompilerParams(
            dimension_semantics=("parallel",),
            vmem_limit_bytes=48 * 1024 * 1024),
    )(flat, wfc, bfc, wh, bh)
    return logp, val


# speedup vs baseline: 1.0394x; 1.0394x over previous
import jax
import jax.numpy as jnp
from jax.experimental import pallas as pl
from jax.experimental.pallas import tpu as pltpu

_STATE = 4
_ACT = 6
_W = 21
_C0, _C1, _C2, _C3 = 64, 32, 64, 64
_M1 = 19 * _W + 20
_M2 = 8 * _W + 9
_S = 4
_G = 8


def _feat_kernel(x_ref, w1_ref, b1_ref, w2_ref, b2_ref, w3_ref, b3_ref,
                 o_ref, xp_ref, y1_ref, buf2_ref, y2_ref, buf3_ref):
    for g in range(_G):
        for s in range(_S):
            xp_ref[g, :, pl.ds(s * _C0, _C0)] = x_ref[g * _S + s]
    for g in range(_G):
        acc = None
        for p in range(2):
            for q in range(2):
                xs = xp_ref[g, pl.ds(p * _W + q, _M1), :]
                c = jnp.dot(xs, w1_ref[p * 2 + q],
                            preferred_element_type=jnp.float32)
                acc = c if acc is None else acc + c
        y1_ref[g] = jnp.maximum(acc + b1_ref[...], 0.0)

        for p in range(4):
            for q in range(4):
                t = p * 4 + q
                buf2_ref[g, :, pl.ds(t * _S * _C1, _S * _C1)] = (
                    y1_ref[g, pl.ds(p * _W + q, _M2, stride=2), :]
                    .astype(jnp.bfloat16))
        y2_ref[g] = jnp.maximum(
            jnp.dot(buf2_ref[g], w2_ref[...],
                    preferred_element_type=jnp.float32) + b2_ref[...], 0.0)

        for p in range(3):
            for q in range(3):
                t = p * 3 + q
                for oh in range(7):
                    buf3_ref[g, pl.ds(oh * 7, 7),
                             pl.ds(t * _S * _C2, _S * _C2)] = (
                        y2_ref[g, pl.ds((oh + p) * _W + q, 7), :]
                        .astype(jnp.bfloat16))
        z = jnp.maximum(
            jnp.dot(buf3_ref[g], w3_ref[...],
                    preferred_element_type=jnp.float32) + b3_ref[...],
            0.0).astype(jnp.bfloat16)
        for s in range(_S):
            o_ref[g * _S + s] = z[:, s * _C3:(s + 1) * _C3]


def _head_kernel(x_ref, wfc_ref, bfc_ref, wh_ref, bh_ref, logp_ref, val_ref):
    feat = jnp.dot(x_ref[...], wfc_ref[...],
                   preferred_element_type=jnp.float32)
    feat = jnp.maximum(feat + bfc_ref[...], 0.0)
    heads = jnp.dot(feat, wh_ref[...],
                    preferred_element_type=jnp.float32) + bh_ref[...]
    logits = heads[:, :_ACT]
    m = jnp.max(logits, axis=1, keepdims=True)
    z = logits - m
    lse = jnp.log(jnp.sum(jnp.exp(z), axis=1, keepdims=True))
    logp_ref[...] = z - lse
    val_ref[...] = heads[:, _ACT:]


def _block_diag(w):
    taps, cin, cout = w.shape
    eye = jnp.eye(_S, dtype=w.dtype)
    wbd = w[:, None, :, None, :] * eye[None, :, None, :, None]
    return wbd.reshape(taps * _S * cin, _S * cout)


def kernel(w1, b1, w2, b2, w3, b3, wfc, bfc, wh, bh, x):
    n = x.shape[0]
    xb = x.astype(jnp.bfloat16)
    x1 = xb.reshape(n, _STATE, _W, 4, _W, 4).transpose(0, 2, 4, 1, 3, 5)
    x1 = x1.reshape(n, _W * _W, _C0)

    w1bd = _block_diag(w1).reshape(4, _S * _C0, _S * _C1).astype(jnp.bfloat16)
    w2bd = _block_diag(w2).astype(jnp.bfloat16)
    w3bd = _block_diag(w3).astype(jnp.bfloat16)
    b1t = jnp.tile(b1, (1, _S))
    b2t = jnp.tile(b2, (1, _S))
    b3t = jnp.tile(b3, (1, _S))

    feat = pl.pallas_call(
        _feat_kernel,
        out_shape=jax.ShapeDtypeStruct((n, 49, _C3), jnp.bfloat16),
        grid=(n // (_G * _S),),
        in_specs=[
            pl.BlockSpec((_G * _S, _W * _W, _C0), lambda i: (i, 0, 0)),
            pl.BlockSpec((4, _S * _C0, _S * _C1), lambda i: (0, 0, 0)),
            pl.BlockSpec((1, _S * _C1), lambda i: (0, 0)),
            pl.BlockSpec((16 * _S * _C1, _S * _C2), lambda i: (0, 0)),
            pl.BlockSpec((1, _S * _C2), lambda i: (0, 0)),
            pl.BlockSpec((9 * _S * _C2, _S * _C3), lambda i: (0, 0)),
            pl.BlockSpec((1, _S * _C3), lambda i: (0, 0)),
        ],
        out_specs=pl.BlockSpec((_G * _S, 49, _C3), lambda i: (i, 0, 0)),
        scratch_shapes=[
            pltpu.VMEM((_G, _W * _W, _S * _C0), jnp.bfloat16),
            pltpu.VMEM((_G, _M1, _S * _C1), jnp.float32),
            pltpu.VMEM((_G, _M2, 16 * _S * _C1), jnp.bfloat16),
            pltpu.VMEM((_G, _M2, _S * _C2), jnp.float32),
            pltpu.VMEM((_G, 49, 9 * _S * _C2), jnp.bfloat16),
        ],
        compiler_params=pltpu.CompilerParams(
            dimension_semantics=("parallel",),
            vmem_limit_bytes=48 * 1024 * 1024),
    )(x1, w1bd, b1t, w2bd, b2t, w3bd, b3t)

    flat = feat.reshape(n, 49 * _C3)

    h = wfc.shape[1]
    a1 = wh.shape[1]
    tm = 128
    logp, val = pl.pallas_call(
        _head_kernel,
        out_shape=(jax.ShapeDtypeStruct((n, a1 - 1), jnp.float32),
                   jax.ShapeDtypeStruct((n, 1), jnp.float32)),
        grid=(pl.cdiv(n, tm),),
        in_specs=[
            pl.BlockSpec((tm, 49 * _C3), lambda i: (i, 0)),
            pl.BlockSpec((49 * _C3, h), lambda i: (0, 0)),
            pl.BlockSpec((1, h), lambda i: (0, 0)),
            pl.BlockSpec((h, a1), lambda i: (0, 0)),
            pl.BlockSpec((1, a1), lambda i: (0, 0)),
        ],
        out_specs=(pl.BlockSpec((tm, a1 - 1), lambda i: (i, 0)),
                   pl.BlockSpec((tm, 1), lambda i: (i, 0))),
        compiler_params=pltpu.CompilerParams(
            dimension_semantics=("parallel",),
            vmem_limit_bytes=48 * 1024 * 1024),
    )(flat, wfc.astype(jnp.bfloat16), bfc, wh, bh)
    return logp, val
